# baseline (speedup 1.0000x reference)
import functools

import jax
import jax.numpy as jnp
from jax import lax
from jax.experimental import pallas as pl
from jax.experimental.pallas import tpu as pltpu
from jax.experimental.pallas import tpu_sc as plsc

N, E, H = 10000, 320000, 128

NC, NS, L = 2, 16, 16

EPT = E // NS
K = 80
NCHUNK = EPT // K
CPG = 50
NGRP = NCHUNK // CPG
ROW_CHUNKS = N // K


_MM_BLOCK = 1000


def _matmul_body(x_ref, uw_ref, ub_ref, vw_ref, vb_ref, ux_ref, vx_ref):
    x_blk = x_ref[...]
    dn = (((1,), (1,)), ((), ()))
    ux_ref[...] = lax.dot_general(
        x_blk, uw_ref[...], dn, precision=lax.Precision.HIGHEST,
        preferred_element_type=jnp.float32) + ub_ref[...][None, :]
    vx_ref[...] = lax.dot_general(
        x_blk, vw_ref[...], dn, precision=lax.Precision.HIGHEST,
        preferred_element_type=jnp.float32) + vb_ref[...][None, :]


def _matmuls(x, Us_W, Us_b, Vs_W, Vs_b):
    grid = (N // _MM_BLOCK,)
    out_shape = jax.ShapeDtypeStruct((N, H), jnp.float32)
    full = lambda i: (0, 0)
    return pl.pallas_call(
        _matmul_body,
        grid=grid,
        in_specs=[
            pl.BlockSpec((_MM_BLOCK, H), lambda i: (i, 0)),
            pl.BlockSpec((H, H), full),
            pl.BlockSpec((H,), lambda i: (0,)),
            pl.BlockSpec((H, H), full),
            pl.BlockSpec((H,), lambda i: (0,)),
        ],
        out_specs=[
            pl.BlockSpec((_MM_BLOCK, H), lambda i: (i, 0)),
            pl.BlockSpec((_MM_BLOCK, H), lambda i: (i, 0)),
        ],
        out_shape=[out_shape, out_shape],
    )(x, Us_W, Us_b, Vs_W, Vs_b)


def _sc_body(vx_hbm, gate_hbm, src_hbm, dst_hbm, agg_hbm, gsum_hbm,
             src_grp, dst_grp, dstbuf, rows_v, gate_v, sem_g, sem_l, sem_sc,
             acc_sh):
    cid = lax.axis_index("c")
    sid = lax.axis_index("s")

    if True:
        zeros = jnp.zeros((L,), jnp.float32)

        @pl.loop(0, K)
        def _(r):
            for j in range(H // L):
                rows_v[0][r, pl.ds(j * L, L)] = zeros

        for k in range(8):
            i = sid + k * NS

            @pl.when(i < ROW_CHUNKS)
            def _():
                pltpu.sync_copy(rows_v[0], acc_sh.at[pl.ds(i * K, K)])

        plsc.subcore_barrier()

        base0 = sid * EPT

        def run_pipeline(with_gather):
            @pl.loop(0, NGRP)
            def _(g):
                gbase = base0 + g * CPG * K

                pltpu.sync_copy(dst_hbm.at[pl.ds(gbase, CPG * K)], dst_grp)
                if with_gather:
                    pltpu.sync_copy(src_hbm.at[pl.ds(gbase, CPG * K)], src_grp)

                def issue_inputs(c, b):
                    pltpu.async_copy(gate_hbm.at[pl.ds(gbase + c * K, K)],
                                     gate_v[b], sem_l[b])
                    if with_gather:
                        pltpu.async_copy(
                            vx_hbm.at[src_grp.at[pl.ds(c * K, K)]],
                            rows_v[b], sem_g[b])

                def drain_gate(c, b):
                    pltpu.make_async_copy(
                        gate_hbm.at[pl.ds(gbase + c * K, K)],
                        gate_v[b], sem_l[b]).wait()

                issue_inputs(0, 0)

                @pl.loop(0, CPG // 2)
                def _(i):
                    for b in range(2):
                        c = 2 * i + b
                        if with_gather:
                            pltpu.make_async_copy(
                                vx_hbm.at[src_grp.at[pl.ds(c * K, K)]],
                                rows_v[b], sem_g[b]).wait()
                        else:
                            drain_gate(c, b)

                        @pl.when(c >= 1)
                        def _():
                            pltpu.make_async_copy(
                                gate_v[1 - b], acc_sh.at[dstbuf[1 - b]],
                                sem_sc[1 - b]).wait()

                        if b == 0:
                            issue_inputs(c + 1, 1)
                        else:
                            @pl.when(i < CPG // 2 - 1)
                            def _():
                                issue_inputs(c + 1, 0)

                        for j in range(K // L):
                            dstbuf[b][pl.ds(j * L, L)] = (
                                dst_grp[pl.ds(c * K + j * L, L)])

                        if with_gather:
                            drain_gate(c, b)

                            @pl.loop(0, K, unroll=2)
                            def _(r):
                                for j in range(H // L):
                                    sl = pl.ds(j * L, L)
                                    gate_v[b][r, sl] = (
                                        rows_v[b][r, sl] * gate_v[b][r, sl])

                        pltpu.async_copy(gate_v[b], acc_sh.at[dstbuf[b]],
                                         sem_sc[b], add=True)

                pltpu.make_async_copy(
                    gate_v[1], acc_sh.at[dstbuf[1]], sem_sc[1]).wait()

        @pl.when(cid == 0)
        def _():
            run_pipeline(True)

        @pl.when(cid == 1)
        def _():
            run_pipeline(False)

        plsc.subcore_barrier()

        for k in range(10):
            b = k % 2
            i = sid + k * NS
            ip = i - 2 * NS
            if k >= 2:
                @pl.when(ip < ROW_CHUNKS)
                def _():
                    slp = pl.ds(ip * K, K)

                    @pl.when(cid == 0)
                    def _():
                        pltpu.make_async_copy(
                            rows_v[b], agg_hbm.at[slp], sem_g[b]).wait()

                    @pl.when(cid == 1)
                    def _():
                        pltpu.make_async_copy(
                            rows_v[b], gsum_hbm.at[slp], sem_g[b]).wait()
            if k < 8:
                @pl.when(i < ROW_CHUNKS)
                def _():
                    sl = pl.ds(i * K, K)
                    pltpu.sync_copy(acc_sh.at[sl], rows_v[b])

                    @pl.when(cid == 0)
                    def _():
                        pltpu.async_copy(rows_v[b], agg_hbm.at[sl], sem_g[b])

                    @pl.when(cid == 1)
                    def _():
                        pltpu.async_copy(rows_v[b], gsum_hbm.at[sl], sem_g[b])


def _sc_aggregate(Vx, edge_gate, src, dst):
    mesh = plsc.VectorSubcoreMesh(core_axis_name="c", subcore_axis_name="s")
    out_t = jax.ShapeDtypeStruct((N, H), jnp.float32)
    kfn = pl.kernel(
        _sc_body,
        out_type=(out_t, out_t),
        mesh=mesh,
        scratch_types=[
            pltpu.VMEM((CPG * K,), jnp.int32),
            pltpu.VMEM((CPG * K,), jnp.int32),
            [pltpu.VMEM((K,), jnp.int32) for _ in range(2)],
            [pltpu.VMEM((K, H), jnp.float32) for _ in range(2)],
            [pltpu.VMEM((K, H), jnp.float32) for _ in range(2)],
            [pltpu.SemaphoreType.DMA for _ in range(2)],
            [pltpu.SemaphoreType.DMA for _ in range(2)],
            [pltpu.SemaphoreType.DMA for _ in range(2)],
            pltpu.VMEM_SHARED((N, H), jnp.float32),
        ],
    )
    return kfn(Vx, edge_gate, src, dst)


def _combine_body(ux_ref, agg_ref, gsum_ref, out_ref):
    out_ref[...] = ux_ref[...] + agg_ref[...] / (gsum_ref[...] + 1e-20)


def _combine(Ux, agg, gsum):
    spec = pl.BlockSpec((_MM_BLOCK, H), lambda i: (i, 0))
    return pl.pallas_call(
        _combine_body,
        grid=(N // _MM_BLOCK,),
        in_specs=[spec, spec, spec],
        out_specs=spec,
        out_shape=jax.ShapeDtypeStruct((N, H), jnp.float32),
    )(Ux, agg, gsum)


def kernel(x, edge_gate, edge_index, Us_W, Us_b, Vs_W, Vs_b):
    Ux, Vx = _matmuls(x, Us_W, Us_b, Vs_W, Vs_b)
    src = edge_index[0]
    dst = edge_index[1]
    agg, gsum = _sc_aggregate(Vx, edge_gate, src, dst)
    return _combine(Ux, agg, gsum)


# speedup vs baseline: 2.0488x; 2.0488x over previous
import functools

import jax
import jax.numpy as jnp
from jax import lax
from jax.experimental import pallas as pl
from jax.experimental.pallas import tpu as pltpu
from jax.experimental.pallas import tpu_sc as plsc

N, E, H = 10000, 320000, 128

NC, NS, L = 2, 16, 16

EPT = E // NS
K = 80
NCHUNK = EPT // K
CPG = 50
NGRP = NCHUNK // CPG
ROW_CHUNKS = N // K


_MM_BLOCK = 1000


def _matmul_body(x_ref, uw_ref, ub_ref, vw_ref, vb_ref, ux_ref, vx_ref):
    x_blk = x_ref[...]
    dn = (((1,), (1,)), ((), ()))
    ux_ref[...] = lax.dot_general(
        x_blk, uw_ref[...], dn, precision=lax.Precision.HIGHEST,
        preferred_element_type=jnp.float32) + ub_ref[...][None, :]
    vx_ref[...] = lax.dot_general(
        x_blk, vw_ref[...], dn, precision=lax.Precision.HIGHEST,
        preferred_element_type=jnp.float32) + vb_ref[...][None, :]


def _matmuls(x, Us_W, Us_b, Vs_W, Vs_b):
    grid = (N // _MM_BLOCK,)
    out_shape = jax.ShapeDtypeStruct((N, H), jnp.float32)
    full = lambda i: (0, 0)
    return pl.pallas_call(
        _matmul_body,
        grid=grid,
        in_specs=[
            pl.BlockSpec((_MM_BLOCK, H), lambda i: (i, 0)),
            pl.BlockSpec((H, H), full),
            pl.BlockSpec((H,), lambda i: (0,)),
            pl.BlockSpec((H, H), full),
            pl.BlockSpec((H,), lambda i: (0,)),
        ],
        out_specs=[
            pl.BlockSpec((_MM_BLOCK, H), lambda i: (i, 0)),
            pl.BlockSpec((_MM_BLOCK, H), lambda i: (i, 0)),
        ],
        out_shape=[out_shape, out_shape],
    )(x, Us_W, Us_b, Vs_W, Vs_b)


def _sc_body(vx_hbm, gate_hbm, src_hbm, dst_hbm, agg_hbm, gsum_hbm,
             src_grp, dst_grp, dstbuf, rows_v, gate_v, sem_g, sem_l, sem_sc,
             acc_sh):
    cid = lax.axis_index("c")
    sid = lax.axis_index("s")

    if True:
        zeros = jnp.zeros((L,), jnp.float32)

        @pl.loop(0, K)
        def _(r):
            for j in range(H // L):
                rows_v[0][r, pl.ds(j * L, L)] = zeros

        for k in range(8):
            i = sid + k * NS

            @pl.when(i < ROW_CHUNKS)
            def _():
                pltpu.sync_copy(rows_v[0], acc_sh.at[pl.ds(i * K, K)])

        plsc.subcore_barrier()

        base0 = sid * EPT

        def run_pipeline(with_gather):
            @pl.loop(0, NGRP)
            def _(g):
                gbase = base0 + g * CPG * K

                pltpu.sync_copy(dst_hbm.at[pl.ds(gbase, CPG * K)], dst_grp)
                if with_gather:
                    pltpu.sync_copy(src_hbm.at[pl.ds(gbase, CPG * K)], src_grp)

                def issue_inputs(c, b):
                    pltpu.async_copy(gate_hbm.at[pl.ds(gbase + c * K, K)],
                                     gate_v[b], sem_l[b])
                    if with_gather:
                        pltpu.async_copy(
                            vx_hbm.at[src_grp.at[pl.ds(c * K, K)]],
                            rows_v[b], sem_g[b])

                def drain_gate(c, b):
                    pltpu.make_async_copy(
                        gate_hbm.at[pl.ds(gbase + c * K, K)],
                        gate_v[b], sem_l[b]).wait()

                issue_inputs(0, 0)

                @pl.loop(0, CPG // 2)
                def _(i):
                    for b in range(2):
                        c = 2 * i + b
                        if with_gather:
                            pltpu.make_async_copy(
                                vx_hbm.at[src_grp.at[pl.ds(c * K, K)]],
                                rows_v[b], sem_g[b]).wait()
                        else:
                            drain_gate(c, b)

                        @pl.when(c >= 1)
                        def _():
                            pltpu.make_async_copy(
                                gate_v[1 - b], acc_sh.at[dstbuf[1 - b]],
                                sem_sc[1 - b]).wait()

                        if b == 0:
                            issue_inputs(c + 1, 1)
                        else:
                            @pl.when(i < CPG // 2 - 1)
                            def _():
                                issue_inputs(c + 1, 0)

                        for j in range(K // L):
                            dstbuf[b][pl.ds(j * L, L)] = (
                                dst_grp[pl.ds(c * K + j * L, L)])

                        if with_gather:
                            drain_gate(c, b)

                            @pl.loop(0, K)
                            def _(r):
                                for j in range(H // L):
                                    sl = pl.ds(j * L, L)
                                    gate_v[b][r, sl] = (
                                        rows_v[b][r, sl] * gate_v[b][r, sl])

                        pltpu.async_copy(gate_v[b], acc_sh.at[dstbuf[b]],
                                         sem_sc[b], add=True)

                pltpu.make_async_copy(
                    gate_v[1], acc_sh.at[dstbuf[1]], sem_sc[1]).wait()

        @pl.when(cid == 0)
        def _():
            run_pipeline(True)

        @pl.when(cid == 1)
        def _():
            run_pipeline(False)

        plsc.subcore_barrier()

        for k in range(10):
            b = k % 2
            i = sid + k * NS
            ip = i - 2 * NS
            if k >= 2:
                @pl.when(ip < ROW_CHUNKS)
                def _():
                    slp = pl.ds(ip * K, K)

                    @pl.when(cid == 0)
                    def _():
                        pltpu.make_async_copy(
                            rows_v[b], agg_hbm.at[slp], sem_g[b]).wait()

                    @pl.when(cid == 1)
                    def _():
                        pltpu.make_async_copy(
                            rows_v[b], gsum_hbm.at[slp], sem_g[b]).wait()
            if k < 8:
                @pl.when(i < ROW_CHUNKS)
                def _():
                    sl = pl.ds(i * K, K)
                    pltpu.sync_copy(acc_sh.at[sl], rows_v[b])

                    @pl.when(cid == 0)
                    def _():
                        pltpu.async_copy(rows_v[b], agg_hbm.at[sl], sem_g[b])

                    @pl.when(cid == 1)
                    def _():
                        pltpu.async_copy(rows_v[b], gsum_hbm.at[sl], sem_g[b])


def _sc_aggregate(Vx, edge_gate, src, dst):
    mesh = plsc.VectorSubcoreMesh(core_axis_name="c", subcore_axis_name="s")
    out_t = jax.ShapeDtypeStruct((N, H), jnp.float32)
    kfn = pl.kernel(
        _sc_body,
        out_type=(out_t, out_t),
        mesh=mesh,
        scratch_types=[
            pltpu.VMEM((CPG * K,), jnp.int32),
            pltpu.VMEM((CPG * K,), jnp.int32),
            [pltpu.VMEM((K,), jnp.int32) for _ in range(2)],
            [pltpu.VMEM((K, H), jnp.float32) for _ in range(2)],
            [pltpu.VMEM((K, H), jnp.float32) for _ in range(2)],
            [pltpu.SemaphoreType.DMA for _ in range(2)],
            [pltpu.SemaphoreType.DMA for _ in range(2)],
            [pltpu.SemaphoreType.DMA for _ in range(2)],
            pltpu.VMEM_SHARED((N, H), jnp.float32),
        ],
    )
    return kfn(Vx, edge_gate, src, dst)


def _combine_body(ux_ref, agg_ref, gsum_ref, out_ref):
    out_ref[...] = ux_ref[...] + agg_ref[...] / (gsum_ref[...] + 1e-20)


def _combine(Ux, agg, gsum):
    spec = pl.BlockSpec((_MM_BLOCK, H), lambda i: (i, 0))
    return pl.pallas_call(
        _combine_body,
        grid=(N // _MM_BLOCK,),
        in_specs=[spec, spec, spec],
        out_specs=spec,
        out_shape=jax.ShapeDtypeStruct((N, H), jnp.float32),
    )(Ux, agg, gsum)


def kernel(x, edge_gate, edge_index, Us_W, Us_b, Vs_W, Vs_b):
    Ux, Vx = _matmuls(x, Us_W, Us_b, Vs_W, Vs_b)
    src = edge_index[0]
    dst = edge_index[1]
    agg, gsum = _sc_aggregate(Vx, edge_gate, src, dst)
    return _combine(Ux, agg, gsum)
